# Initial kernel scaffold; baseline (speedup 1.0000x reference)
import functools

import jax
import jax.numpy as jnp
from jax import lax
from jax.experimental import pallas as pl
from jax.experimental.pallas import tpu as pltpu
from jax.experimental.pallas import tpu_sc as plsc

N = 10000
E = 320000
D = 128

NC = 2
NS = 16
NW = NC * NS

CHUNK = 128
CPT = 79
EPT = CPT * CHUNK
EPAD = NW * EPT
ACC_ROWS = N + 16
HIST = 10240
ZROWS = 125
ROWS_PT = N // NS

_mesh = plsc.VectorSubcoreMesh(core_axis_name="c", subcore_axis_name="s")


@functools.partial(
    pl.kernel,
    out_type=jax.ShapeDtypeStruct((NW * HIST,), jnp.float32),
    mesh=_mesh,
    scratch_types=[
        pltpu.VMEM((EPT,), jnp.int32),
        pltpu.VMEM((HIST,), jnp.float32),
    ],
)
def _degree_hist(dst_hbm, out_hbm, dstv, hist):
    c = lax.axis_index("c")
    s = lax.axis_index("s")
    wid = c * NS + s
    pltpu.sync_copy(dst_hbm.at[pl.ds(wid * EPT, EPT)], dstv)

    @pl.loop(0, HIST, step=16)
    def _(i):
        hist[pl.ds(i, 16)] = jnp.zeros((16,), jnp.float32)

    ones = jnp.ones((16,), jnp.float32)

    @pl.loop(0, EPT, step=16)
    def _(k):
        plsc.addupdate_scatter(hist, [dstv[pl.ds(k, 16)]], ones)

    pltpu.sync_copy(hist, out_hbm.at[pl.ds(wid * HIST, HIST)])


@functools.partial(
    pl.kernel,
    out_type=jax.ShapeDtypeStruct((NC, N, D), jnp.float32),
    mesh=_mesh,
    scratch_types=[
        pltpu.VMEM((CPT, CHUNK), jnp.int32),
        pltpu.VMEM((CPT, CHUNK), jnp.int32),
        pltpu.VMEM((CHUNK, D), jnp.float32),
        pltpu.VMEM((ZROWS, D), jnp.float32),
        pltpu.VMEM_SHARED((ACC_ROWS, D), jnp.float32),
    ],
)
def _propagate_rows(hp_hbm, src_hbm, dst_hbm, out_hbm, srcv, dstv, buf, zbuf, acc):
    c = lax.axis_index("c")
    s = lax.axis_index("s")
    wid = c * NS + s

    pltpu.sync_copy(src_hbm.at[pl.ds(wid * CPT, CPT)], srcv)
    pltpu.sync_copy(dst_hbm.at[pl.ds(wid * CPT, CPT)], dstv)

    @pl.loop(0, ZROWS)
    def _(r):
        @pl.loop(0, D, step=16)
        def _(k):
            zbuf[r, pl.ds(k, 16)] = jnp.zeros((16,), jnp.float32)

    row0 = s * ROWS_PT

    @pl.loop(0, 5)
    def _(i):
        pltpu.sync_copy(zbuf, acc.at[pl.ds(row0 + i * ZROWS, ZROWS)])

    plsc.subcore_barrier()

    @pl.loop(0, CPT)
    def _(j):
        pltpu.sync_copy(hp_hbm.at[srcv.at[j]], buf)
        pltpu.sync_copy(buf, acc.at[dstv.at[j]], add=True)

    plsc.subcore_barrier()
    pltpu.sync_copy(acc.at[pl.ds(row0, ROWS_PT)],
                    out_hbm.at[c, pl.ds(row0, ROWS_PT)])


@functools.partial(
    pl.kernel,
    out_type=jax.ShapeDtypeStruct((NW * HIST,), jnp.float32),
    mesh=_mesh,
    scratch_types=[
        pltpu.VMEM((EPT,), jnp.int32),
        pltpu.VMEM((EPT,), jnp.int32),
        pltpu.VMEM((N,), jnp.float32),
        pltpu.VMEM((HIST,), jnp.float32),
    ],
)
def _propagate_scalar(h2p_hbm, src_hbm, dst_hbm, out_hbm, srcv, dstv, table, hist):
    c = lax.axis_index("c")
    s = lax.axis_index("s")
    wid = c * NS + s
    pltpu.sync_copy(src_hbm.at[pl.ds(wid * EPT, EPT)], srcv)
    pltpu.sync_copy(dst_hbm.at[pl.ds(wid * EPT, EPT)], dstv)
    pltpu.sync_copy(h2p_hbm, table)

    @pl.loop(0, HIST, step=16)
    def _(i):
        hist[pl.ds(i, 16)] = jnp.zeros((16,), jnp.float32)

    @pl.loop(0, EPT, step=16)
    def _(k):
        vals = plsc.load_gather(table, [srcv[pl.ds(k, 16)]])
        plsc.addupdate_scatter(hist, [dstv[pl.ds(k, 16)]], vals)

    pltpu.sync_copy(hist, out_hbm.at[pl.ds(wid * HIST, HIST)])


def _tc_prep_body(x_ref, w1_ref, degc_ref, hp_ref, dinv_ref):
    deg = 1.0 + jnp.sum(degc_ref[...], axis=1, keepdims=True)
    dinv = lax.rsqrt(jnp.maximum(deg, 1e-12))
    h = jnp.dot(x_ref[...], w1_ref[...], preferred_element_type=jnp.float32)
    hp_ref[...] = h * dinv
    dinv_ref[...] = dinv


_tc_prep = pl.pallas_call(
    _tc_prep_body,
    out_shape=(jax.ShapeDtypeStruct((N, D), jnp.float32),
               jax.ShapeDtypeStruct((N, 1), jnp.float32)),
)


def _tc_mid_body(a0_ref, a1_ref, hp_ref, dinv_ref, b1_ref, w2_ref, h2p_ref):
    dinv = dinv_ref[...]
    z = dinv * (a0_ref[...] + a1_ref[...] + hp_ref[...]) + b1_ref[...]
    z = jnp.maximum(z, 0.0)
    h2 = jnp.dot(z, w2_ref[...], preferred_element_type=jnp.float32)
    h2p_ref[...] = h2 * dinv


_tc_mid = pl.pallas_call(
    _tc_mid_body,
    out_shape=jax.ShapeDtypeStruct((N, 1), jnp.float32),
)


def _tc_out_body(acc2c_ref, h2p_ref, dinv_ref, b2_ref, out_ref):
    ssum = jnp.sum(acc2c_ref[...], axis=1, keepdims=True)
    out_ref[...] = jax.nn.sigmoid(dinv_ref[...] * (ssum + h2p_ref[...])
                                  + b2_ref[...])


_tc_out = pl.pallas_call(
    _tc_out_body,
    out_shape=jax.ShapeDtypeStruct((N, 1), jnp.float32),
)


def kernel(x, edge_index, W1, b1, W2, b2):
    src = edge_index[0]
    dst = edge_index[1]

    pad_ids = lax.iota(jnp.int32, EPAD - E)
    src_p = jnp.concatenate([src, pad_ids % N])
    dst_p = jnp.concatenate([dst, N + (pad_ids % 16)])
    src2d = src_p.reshape(NW * CPT, CHUNK)
    dst2d = dst_p.reshape(NW * CPT, CHUNK)

    deg_flat = _degree_hist(dst_p)
    degc = deg_flat.reshape(NW, HIST)[:, :N].T

    hp, dinv = _tc_prep(x, W1, degc)
    accs = _propagate_rows(hp, src2d, dst2d)
    h2p = _tc_mid(accs[0], accs[1], hp, dinv, b1.reshape(1, D), W2)

    acc2_flat = _propagate_scalar(h2p.reshape(N), src_p, dst_p)
    acc2c = acc2_flat.reshape(NW, HIST)[:, :N].T

    return _tc_out(acc2c, h2p, dinv, b2.reshape(1, 1))


# speedup vs baseline: 35.9175x; 35.9175x over previous
import dataclasses
import functools

import jax
import jax.numpy as jnp
from jax import lax
from jax.experimental import pallas as pl
from jax.experimental.pallas import tpu as pltpu
from jax.experimental.pallas import tpu_sc as plsc

N = 10000
E = 320000
D = 128

NC = 2
NS = 16
NW = NC * NS

CHUNK = 128
CPT = 80
EPT = CPT * CHUNK
EPAD = NW * EPT
ACC_ROWS = N + 16
HIST = 10240
ROWS_A = 632
ROWS_LAST = N - 15 * ROWS_A
ZROWS_LAST = ACC_ROWS - 15 * ROWS_A

_mesh = plsc.VectorSubcoreMesh(core_axis_name="c", subcore_axis_name="s")

_sc_params = pltpu.CompilerParams()
if "needs_layout_passes" in pltpu.CompilerParams.__dataclass_fields__:
    _sc_params = dataclasses.replace(_sc_params, needs_layout_passes=False)


@functools.partial(
    pl.kernel,
    out_type=jax.ShapeDtypeStruct((NW * HIST,), jnp.float32),
    mesh=_mesh,
    scratch_types=[
        pltpu.VMEM((EPT,), jnp.int32),
        pltpu.VMEM((HIST,), jnp.float32),
    ],
    compiler_params=_sc_params,
)
def _degree_hist(dst_hbm, out_hbm, dstv, hist):
    c = lax.axis_index("c")
    s = lax.axis_index("s")
    wid = c * NS + s
    pltpu.sync_copy(dst_hbm.at[pl.ds(wid * EPT, EPT)], dstv)

    @pl.loop(0, HIST, step=16)
    def _(i):
        hist[pl.ds(i, 16)] = jnp.zeros((16,), jnp.float32)

    ones = jnp.ones((16,), jnp.float32)

    @pl.loop(0, EPT, step=16)
    def _(k):
        plsc.addupdate_scatter(hist, [dstv[pl.ds(k, 16)]], ones)

    pltpu.sync_copy(hist, out_hbm.at[pl.ds(wid * HIST, HIST)])


@functools.partial(
    pl.kernel,
    out_type=jax.ShapeDtypeStruct((NC, N, D), jnp.float32),
    mesh=_mesh,
    scratch_types=[
        pltpu.VMEM((CPT, CHUNK), jnp.int32),
        pltpu.VMEM((CPT, CHUNK), jnp.int32),
        pltpu.VMEM((CHUNK, D), jnp.float32),
        pltpu.VMEM_SHARED((ACC_ROWS, D), jnp.float32),
    ],
)
def _propagate_rows(hp_hbm, src_hbm, dst_hbm, zeros_hbm, out_hbm,
                    srcv, dstv, buf, acc):
    c = lax.axis_index("c")
    s = lax.axis_index("s")
    wid = c * NS + s

    pltpu.sync_copy(src_hbm.at[pl.ds(wid * CPT, CPT)], srcv)
    pltpu.sync_copy(dst_hbm.at[pl.ds(wid * CPT, CPT)], dstv)

    row0 = s * ROWS_A

    @pl.when(s < NS - 1)
    def _():
        pltpu.sync_copy(zeros_hbm.at[pl.ds(row0, ROWS_A)],
                        acc.at[pl.ds(row0, ROWS_A)])

    @pl.when(s == NS - 1)
    def _():
        pltpu.sync_copy(zeros_hbm.at[pl.ds(row0, ZROWS_LAST)],
                        acc.at[pl.ds(row0, ZROWS_LAST)])

    plsc.subcore_barrier()

    @pl.loop(0, CPT)
    def _(j):
        pltpu.sync_copy(hp_hbm.at[srcv.at[j]], buf)
        pltpu.sync_copy(buf, acc.at[dstv.at[j]], add=True)

    plsc.subcore_barrier()

    @pl.when(s < NS - 1)
    def _():
        pltpu.sync_copy(acc.at[pl.ds(row0, ROWS_A)],
                        out_hbm.at[c, pl.ds(row0, ROWS_A)])

    @pl.when(s == NS - 1)
    def _():
        pltpu.sync_copy(acc.at[pl.ds(row0, ROWS_LAST)],
                        out_hbm.at[c, pl.ds(row0, ROWS_LAST)])


@functools.partial(
    pl.kernel,
    out_type=jax.ShapeDtypeStruct((NW * HIST,), jnp.float32),
    mesh=_mesh,
    scratch_types=[
        pltpu.VMEM((EPT,), jnp.int32),
        pltpu.VMEM((EPT,), jnp.int32),
        pltpu.VMEM((N,), jnp.float32),
        pltpu.VMEM((HIST,), jnp.float32),
    ],
    compiler_params=_sc_params,
)
def _propagate_scalar(h2p_hbm, src_hbm, dst_hbm, out_hbm, srcv, dstv, table, hist):
    c = lax.axis_index("c")
    s = lax.axis_index("s")
    wid = c * NS + s
    pltpu.sync_copy(src_hbm.at[pl.ds(wid * EPT, EPT)], srcv)
    pltpu.sync_copy(dst_hbm.at[pl.ds(wid * EPT, EPT)], dstv)
    pltpu.sync_copy(h2p_hbm, table)

    @pl.loop(0, HIST, step=16)
    def _(i):
        hist[pl.ds(i, 16)] = jnp.zeros((16,), jnp.float32)

    @pl.loop(0, EPT, step=16)
    def _(k):
        vals = plsc.load_gather(table, [srcv[pl.ds(k, 16)]])
        plsc.addupdate_scatter(hist, [dstv[pl.ds(k, 16)]], vals)

    pltpu.sync_copy(hist, out_hbm.at[pl.ds(wid * HIST, HIST)])


def _tc_prep_body(x_ref, w1_ref, degc_ref, hp_ref, dinv_ref):
    deg = 1.0 + jnp.sum(degc_ref[...], axis=1, keepdims=True)
    dinv = lax.rsqrt(jnp.maximum(deg, 1e-12))
    h = jnp.dot(x_ref[...], w1_ref[...], preferred_element_type=jnp.float32)
    hp_ref[...] = h * dinv
    dinv_ref[...] = dinv


_tc_prep = pl.pallas_call(
    _tc_prep_body,
    out_shape=(jax.ShapeDtypeStruct((N, D), jnp.float32),
               jax.ShapeDtypeStruct((N, 1), jnp.float32)),
)


def _tc_mid_body(a0_ref, a1_ref, hp_ref, dinv_ref, b1_ref, w2_ref, h2p_ref):
    dinv = dinv_ref[...]
    z = dinv * (a0_ref[...] + a1_ref[...] + hp_ref[...]) + b1_ref[...]
    z = jnp.maximum(z, 0.0)
    h2 = jnp.dot(z, w2_ref[...], preferred_element_type=jnp.float32)
    h2p_ref[...] = h2 * dinv


_tc_mid = pl.pallas_call(
    _tc_mid_body,
    out_shape=jax.ShapeDtypeStruct((N, 1), jnp.float32),
)


def _tc_out_body(acc2c_ref, h2p_ref, dinv_ref, b2_ref, out_ref):
    ssum = jnp.sum(acc2c_ref[...], axis=1, keepdims=True)
    out_ref[...] = jax.nn.sigmoid(dinv_ref[...] * (ssum + h2p_ref[...])
                                  + b2_ref[...])


_tc_out = pl.pallas_call(
    _tc_out_body,
    out_shape=jax.ShapeDtypeStruct((N, 1), jnp.float32),
)


def kernel(x, edge_index, W1, b1, W2, b2):
    src = edge_index[0]
    dst = edge_index[1]

    pad_ids = lax.iota(jnp.int32, EPAD - E)
    src_p = jnp.concatenate([src, pad_ids % N])
    dst_p = jnp.concatenate([dst, N + (pad_ids % 16)])
    src2d = src_p.reshape(NW * CPT, CHUNK)
    dst2d = dst_p.reshape(NW * CPT, CHUNK)

    deg_flat = _degree_hist(dst_p)
    degc = deg_flat.reshape(NW, HIST)[:, :N].T

    hp, dinv = _tc_prep(x, W1, degc)
    zeros = jnp.zeros((ACC_ROWS, D), jnp.float32)
    accs = _propagate_rows(hp, src2d, dst2d, zeros)
    h2p = _tc_mid(accs[0], accs[1], hp, dinv, b1.reshape(1, D), W2)

    acc2_flat = _propagate_scalar(h2p.reshape(N), src_p, dst_p)
    acc2c = acc2_flat.reshape(NW, HIST)[:, :N].T

    return _tc_out(acc2c, h2p, dinv, b2.reshape(1, 1))


# speedup vs baseline: 46.5077x; 1.2948x over previous
import dataclasses
import functools

import jax
import jax.numpy as jnp
from jax import lax
from jax.experimental import pallas as pl
from jax.experimental.pallas import tpu as pltpu
from jax.experimental.pallas import tpu_sc as plsc

N = 10000
E = 320000
D = 128

NC = 2
NS = 16
NW = NC * NS

CHUNK = 128
CPT = 80
PASSES = 2
CPP = CPT // PASSES
EPT = CPT * CHUNK
EPAD = NW * EPT
ACC_ROWS = N + 16
HIST = 10240
ROWS_A = 632
ROWS_LAST = N - 15 * ROWS_A
ZROWS_LAST = ACC_ROWS - 15 * ROWS_A

_mesh = plsc.VectorSubcoreMesh(core_axis_name="c", subcore_axis_name="s")

_sc_params = pltpu.CompilerParams()
if "needs_layout_passes" in pltpu.CompilerParams.__dataclass_fields__:
    _sc_params = dataclasses.replace(_sc_params, needs_layout_passes=False)


@functools.partial(
    pl.kernel,
    out_type=jax.ShapeDtypeStruct((NW * HIST,), jnp.float32),
    mesh=_mesh,
    scratch_types=[
        pltpu.VMEM((EPT,), jnp.int32),
        pltpu.VMEM((HIST,), jnp.float32),
    ],
    compiler_params=_sc_params,
)
def _degree_hist(dst_hbm, out_hbm, dstv, hist):
    c = lax.axis_index("c")
    s = lax.axis_index("s")
    wid = c * NS + s
    pltpu.sync_copy(dst_hbm.at[pl.ds(wid * EPT, EPT)], dstv)

    @pl.loop(0, HIST, step=16)
    def _(i):
        hist[pl.ds(i, 16)] = jnp.zeros((16,), jnp.float32)

    ones = jnp.ones((16,), jnp.float32)

    @pl.loop(0, EPT, step=16)
    def _(k):
        plsc.addupdate_scatter(hist, [dstv[pl.ds(k, 16)]], ones)

    pltpu.sync_copy(hist, out_hbm.at[pl.ds(wid * HIST, HIST)])


@functools.partial(
    pl.kernel,
    out_type=jax.ShapeDtypeStruct((NC, N, D), jnp.float32),
    mesh=_mesh,
    scratch_types=[
        pltpu.VMEM((CPP, CHUNK), jnp.int32),
        pltpu.VMEM((CPP, CHUNK), jnp.int32),
        pltpu.VMEM((CHUNK, D), jnp.float32),
        pltpu.VMEM((CHUNK, D), jnp.float32),
        pltpu.VMEM_SHARED((ACC_ROWS, D), jnp.float32),
        pltpu.SemaphoreType.DMA,
        pltpu.SemaphoreType.DMA,
        pltpu.SemaphoreType.DMA,
        pltpu.SemaphoreType.DMA,
    ],
)
def _propagate_rows(hp_hbm, src_hbm, dst_hbm, zeros_hbm, out_hbm,
                    srcv, dstv, buf0, buf1, acc, sg0, sg1, ss0, ss1):
    c = lax.axis_index("c")
    s = lax.axis_index("s")
    wid = c * NS + s

    row0 = s * ROWS_A

    @pl.when(s < NS - 1)
    def _():
        pltpu.sync_copy(zeros_hbm.at[pl.ds(row0, ROWS_A)],
                        acc.at[pl.ds(row0, ROWS_A)])

    @pl.when(s == NS - 1)
    def _():
        pltpu.sync_copy(zeros_hbm.at[pl.ds(row0, ZROWS_LAST)],
                        acc.at[pl.ds(row0, ZROWS_LAST)])

    plsc.subcore_barrier()

    def _gather(j, buf, sem):
        pltpu.async_copy(hp_hbm.at[srcv.at[j]], buf, sem)

    def _wait_gather(j, buf, sem):
        pltpu.make_async_copy(hp_hbm.at[srcv.at[j]], buf, sem).wait()

    def _scatter(j, buf, sem):
        pltpu.async_copy(buf, acc.at[dstv.at[j]], sem, add=True)

    def _wait_scatter(j, buf, sem):
        pltpu.make_async_copy(buf, acc.at[dstv.at[j]], sem).wait()

    for p in range(PASSES):
        pltpu.sync_copy(src_hbm.at[pl.ds((wid * PASSES + p) * CPP, CPP)], srcv)
        pltpu.sync_copy(dst_hbm.at[pl.ds((wid * PASSES + p) * CPP, CPP)], dstv)

        _gather(0, buf0, sg0)

        @pl.loop(0, CPP, step=2)
        def _(j):
            @pl.when(j > 0)
            def _():
                _wait_scatter(j - 1, buf1, ss1)

            _gather(j + 1, buf1, sg1)
            _wait_gather(j, buf0, sg0)
            _scatter(j, buf0, ss0)
            _wait_scatter(j, buf0, ss0)

            @pl.when(j + 2 < CPP)
            def _():
                _gather(j + 2, buf0, sg0)

            _wait_gather(j + 1, buf1, sg1)
            _scatter(j + 1, buf1, ss1)

        _wait_scatter(CPP - 1, buf1, ss1)

    plsc.subcore_barrier()

    @pl.when(s < NS - 1)
    def _():
        pltpu.sync_copy(acc.at[pl.ds(row0, ROWS_A)],
                        out_hbm.at[c, pl.ds(row0, ROWS_A)])

    @pl.when(s == NS - 1)
    def _():
        pltpu.sync_copy(acc.at[pl.ds(row0, ROWS_LAST)],
                        out_hbm.at[c, pl.ds(row0, ROWS_LAST)])


@functools.partial(
    pl.kernel,
    out_type=jax.ShapeDtypeStruct((NW * HIST,), jnp.float32),
    mesh=_mesh,
    scratch_types=[
        pltpu.VMEM((EPT,), jnp.int32),
        pltpu.VMEM((EPT,), jnp.int32),
        pltpu.VMEM((N,), jnp.float32),
        pltpu.VMEM((HIST,), jnp.float32),
    ],
    compiler_params=_sc_params,
)
def _propagate_scalar(h2p_hbm, src_hbm, dst_hbm, out_hbm, srcv, dstv, table, hist):
    c = lax.axis_index("c")
    s = lax.axis_index("s")
    wid = c * NS + s
    pltpu.sync_copy(src_hbm.at[pl.ds(wid * EPT, EPT)], srcv)
    pltpu.sync_copy(dst_hbm.at[pl.ds(wid * EPT, EPT)], dstv)
    pltpu.sync_copy(h2p_hbm, table)

    @pl.loop(0, HIST, step=16)
    def _(i):
        hist[pl.ds(i, 16)] = jnp.zeros((16,), jnp.float32)

    @pl.loop(0, EPT, step=16)
    def _(k):
        vals = plsc.load_gather(table, [srcv[pl.ds(k, 16)]])
        plsc.addupdate_scatter(hist, [dstv[pl.ds(k, 16)]], vals)

    pltpu.sync_copy(hist, out_hbm.at[pl.ds(wid * HIST, HIST)])


def _tc_prep_body(x_ref, w1_ref, degc_ref, hp_ref, dinv_ref):
    deg = 1.0 + jnp.sum(degc_ref[...], axis=1, keepdims=True)
    dinv = lax.rsqrt(jnp.maximum(deg, 1e-12))
    h = jnp.dot(x_ref[...], w1_ref[...], preferred_element_type=jnp.float32)
    hp_ref[...] = h * dinv
    dinv_ref[...] = dinv


_tc_prep = pl.pallas_call(
    _tc_prep_body,
    out_shape=(jax.ShapeDtypeStruct((N, D), jnp.float32),
               jax.ShapeDtypeStruct((N, 1), jnp.float32)),
)


def _tc_mid_body(a0_ref, a1_ref, hp_ref, dinv_ref, b1_ref, w2_ref, h2p_ref):
    dinv = dinv_ref[...]
    z = dinv * (a0_ref[...] + a1_ref[...] + hp_ref[...]) + b1_ref[...]
    z = jnp.maximum(z, 0.0)
    h2 = jnp.dot(z, w2_ref[...], preferred_element_type=jnp.float32)
    h2p_ref[...] = h2 * dinv


_tc_mid = pl.pallas_call(
    _tc_mid_body,
    out_shape=jax.ShapeDtypeStruct((N, 1), jnp.float32),
)


def _tc_out_body(acc2c_ref, h2p_ref, dinv_ref, b2_ref, out_ref):
    ssum = jnp.sum(acc2c_ref[...], axis=1, keepdims=True)
    out_ref[...] = jax.nn.sigmoid(dinv_ref[...] * (ssum + h2p_ref[...])
                                  + b2_ref[...])


_tc_out = pl.pallas_call(
    _tc_out_body,
    out_shape=jax.ShapeDtypeStruct((N, 1), jnp.float32),
)


def kernel(x, edge_index, W1, b1, W2, b2):
    src = edge_index[0]
    dst = edge_index[1]

    pad_ids = lax.iota(jnp.int32, EPAD - E)
    src_p = jnp.concatenate([src, pad_ids % N])
    dst_p = jnp.concatenate([dst, N + (pad_ids % 16)])
    src2d = src_p.reshape(NW * CPT, CHUNK)
    dst2d = dst_p.reshape(NW * CPT, CHUNK)

    deg_flat = _degree_hist(dst_p)
    degc = deg_flat.reshape(NW, HIST)[:, :N].T

    hp, dinv = _tc_prep(x, W1, degc)
    zeros = jnp.zeros((ACC_ROWS, D), jnp.float32)
    accs = _propagate_rows(hp, src2d, dst2d, zeros)
    h2p = _tc_mid(accs[0], accs[1], hp, dinv, b1.reshape(1, D), W2)

    acc2_flat = _propagate_scalar(h2p.reshape(N), src_p, dst_p)
    acc2c = acc2_flat.reshape(NW, HIST)[:, :N].T

    return _tc_out(acc2c, h2p, dinv, b2.reshape(1, 1))
